# baseline (speedup 1.0000x reference)
import functools

import jax
import jax.numpy as jnp
from jax import lax
from jax.experimental import pallas as pl
from jax.experimental.pallas import tpu as pltpu
from jax.experimental.pallas import tpu_sc as plsc

N_NODES = 10000
N_EDGES = 320000
D = 128

NC = 2
NS = 16
NW = NC * NS

CHUNK = 128
CHUNKS_T = 80
NB = 2
IBLK = 8
SUPER = 2 * IBLK
NSUP = CHUNKS_T // SUPER
E_PAD = NW * CHUNKS_T * CHUNK
N_PAD = 10112
ZROWS = N_PAD // NS


def _sc_aggregate(src2d, dst2d, x):
    mesh = plsc.VectorSubcoreMesh(core_axis_name="c", subcore_axis_name="s")

    @functools.partial(
        pl.kernel,
        out_type=jax.ShapeDtypeStruct((NC * N_PAD, D), jnp.float32),
        mesh=mesh,
        scratch_types=[
            pltpu.VMEM((2, IBLK, CHUNK), jnp.int32),
            pltpu.VMEM((2, IBLK, CHUNK), jnp.int32),
            pltpu.VMEM_SHARED((N_PAD, D), jnp.float32),
            pltpu.SemaphoreType.DMA,
            pltpu.SemaphoreType.DMA,
            pltpu.SemaphoreType.DMA,
            pltpu.SemaphoreType.DMA,
            pltpu.SemaphoreType.DMA,
            pltpu.SemaphoreType.DMA,
        ],
    )
    def sc_kernel(src_hbm, dst_hbm, x_hbm, out_hbm,
                  src_v, dst_v, agg, sem_sc, semi,
                  semg0, semg1, semg2, semg3):
        cid = lax.axis_index("c")
        sid = lax.axis_index("s")
        tid = cid * NS + sid
        semg = (semg0, semg1, semg2, semg3)

        def slot(off):
            return (off % SUPER) // IBLK

        def row(off):
            return off % IBLK

        def body(bufs):
            buf = [bufs.at[u] for u in range(NB)]

            def srow(off):
                return src_v.at[slot(off), row(off)]

            def drow(off):
                return dst_v.at[slot(off), row(off)]

            def start_gather(k, off):
                pltpu.async_copy(x_hbm.at[srow(off)], buf[off % NB],
                                 semg[off % NB])

            def wait_gather(k, off):
                pltpu.make_async_copy(x_hbm.at[srow(off)], buf[off % NB],
                                      semg[off % NB]).wait()

            def start_scatter(k, off):
                pltpu.async_copy(buf[off % NB], agg.at[drow(off)], sem_sc,
                                 add=True)

            def drain_scatter(off):
                pltpu.make_async_copy(buf[off % NB], agg.at[drow(off)],
                                      sem_sc).wait()

            @pl.loop(0, CHUNK)
            def _(i):
                for k in range(D // 16):
                    bufs[0, i, pl.ds(k * 16, 16)] = jnp.zeros((16,),
                                                              jnp.float32)

            zbase = sid * ZROWS
            nfull = ZROWS // CHUNK
            for z in range(nfull):
                pltpu.sync_copy(buf[0], agg.at[pl.ds(zbase + z * CHUNK,
                                                     CHUNK)])
            rem = ZROWS - nfull * CHUNK
            if rem:
                pltpu.sync_copy(buf[0].at[pl.ds(0, rem)],
                                agg.at[pl.ds(zbase + nfull * CHUNK, rem)])
            plsc.subcore_barrier()

            base = tid * CHUNKS_T
            pltpu.sync_copy(src_hbm.at[pl.ds(base, IBLK)], src_v.at[0])
            pltpu.sync_copy(dst_hbm.at[pl.ds(base, IBLK)], dst_v.at[0])
            pltpu.sync_copy(src_hbm.at[pl.ds(base + IBLK, IBLK)], src_v.at[1])
            pltpu.sync_copy(dst_hbm.at[pl.ds(base + IBLK, IBLK)], dst_v.at[1])

            for off in range(NB - 1):
                start_gather(0, off)

            @pl.loop(0, NSUP)
            def _(k):
                for off in range(SUPER):
                    wait_gather(k, off)
                    if off == 0:
                        @pl.when(k > 0)
                        def _():
                            drain_scatter(SUPER - 1)
                    else:
                        drain_scatter(off - 1)
                    start_scatter(k, off)

                    if off == 0:
                        @pl.when(k > 0)
                        def _():
                            nb_ = base + k * SUPER + IBLK
                            pltpu.async_copy(src_hbm.at[pl.ds(nb_, IBLK)],
                                             src_v.at[1], semi)
                            pltpu.async_copy(dst_hbm.at[pl.ds(nb_, IBLK)],
                                             dst_v.at[1], semi)
                    if off == IBLK - NB + 1:
                        @pl.when(k > 0)
                        def _():
                            nb_ = base + k * SUPER + IBLK
                            pltpu.make_async_copy(
                                src_hbm.at[pl.ds(nb_, IBLK)],
                                src_v.at[1], semi).wait()
                            pltpu.make_async_copy(
                                dst_hbm.at[pl.ds(nb_, IBLK)],
                                dst_v.at[1], semi).wait()
                    if off == IBLK:
                        @pl.when(k < NSUP - 1)
                        def _():
                            nb_ = base + (k + 1) * SUPER
                            pltpu.async_copy(src_hbm.at[pl.ds(nb_, IBLK)],
                                             src_v.at[0], semi)
                            pltpu.async_copy(dst_hbm.at[pl.ds(nb_, IBLK)],
                                             dst_v.at[0], semi)
                    if off == SUPER - NB + 1:
                        @pl.when(k < NSUP - 1)
                        def _():
                            nb_ = base + (k + 1) * SUPER
                            pltpu.make_async_copy(
                                src_hbm.at[pl.ds(nb_, IBLK)],
                                src_v.at[0], semi).wait()
                            pltpu.make_async_copy(
                                dst_hbm.at[pl.ds(nb_, IBLK)],
                                dst_v.at[0], semi).wait()

                    off_n = off + NB - 1
                    if off_n < SUPER:
                        start_gather(k, off_n)
                    else:
                        @pl.when(k < NSUP - 1)
                        def _():
                            start_gather(k + 1, off_n - SUPER)

            drain_scatter(SUPER - 1)
            plsc.subcore_barrier()

            obase = sid * ZROWS
            hbase = cid * N_PAD + obase
            nfull_o = ZROWS // CHUNK
            rem_o = ZROWS - nfull_o * CHUNK
            if rem_o:
                rbuf = bufs.at[0, pl.ds(0, rem_o)]
                pltpu.sync_copy(
                    agg.at[pl.ds(obase + nfull_o * CHUNK, rem_o)], rbuf)
                pltpu.async_copy(
                    rbuf, out_hbm.at[pl.ds(hbase + nfull_o * CHUNK, rem_o)],
                    sem_sc)
                pltpu.make_async_copy(
                    rbuf, out_hbm.at[pl.ds(hbase + nfull_o * CHUNK, rem_o)],
                    sem_sc).wait()
            for p in range(nfull_o):
                if p >= NB:
                    pltpu.make_async_copy(
                        buf[p % NB],
                        out_hbm.at[pl.ds(hbase + (p - NB) * CHUNK, CHUNK)],
                        sem_sc).wait()
                pltpu.sync_copy(agg.at[pl.ds(obase + p * CHUNK, CHUNK)],
                                buf[p % NB])
                pltpu.async_copy(buf[p % NB],
                                 out_hbm.at[pl.ds(hbase + p * CHUNK, CHUNK)],
                                 sem_sc)
            for p in range(max(nfull_o - NB, 0), nfull_o):
                pltpu.make_async_copy(
                    buf[p % NB],
                    out_hbm.at[pl.ds(hbase + p * CHUNK, CHUNK)],
                    sem_sc).wait()

        pl.run_scoped(body, pltpu.VMEM((NB, CHUNK, D), jnp.float32))

    return sc_kernel(src2d, dst2d, x)


def _tc_body(x_ref, a0_ref, a1_ref, w_ref, b_ref, o_ref):
    h = x_ref[...] + a0_ref[0] + a1_ref[0]
    o_ref[...] = lax.dot_general(
        h, w_ref[...],
        dimension_numbers=(((1,), (1,)), ((), ())),
        preferred_element_type=jnp.float32,
    ) + b_ref[...]


def kernel(x, edge_index, W, b):
    src = edge_index[0]
    dst = edge_index[1]
    pad = E_PAD - N_EDGES
    pad_i = jnp.arange(pad, dtype=jnp.int32)
    src_p = jnp.concatenate([src, pad_i % N_NODES])
    dst_p = jnp.concatenate([dst, N_NODES + pad_i % (N_PAD - N_NODES)])
    src2d = src_p.reshape(NW * CHUNKS_T, CHUNK)
    dst2d = dst_p.reshape(NW * CHUNKS_T, CHUNK)

    agg = _sc_aggregate(src2d, dst2d, x).reshape(NC, N_PAD, D)

    BM = 1000
    nb = N_NODES // BM
    out = pl.pallas_call(
        _tc_body,
        grid=(nb,),
        in_specs=[
            pl.BlockSpec((BM, D), lambda i: (i, 0)),
            pl.BlockSpec((1, BM, D), lambda i: (0, i, 0)),
            pl.BlockSpec((1, BM, D), lambda i: (1, i, 0)),
            pl.BlockSpec((D, D), lambda i: (0, 0)),
            pl.BlockSpec((1, D), lambda i: (0, 0)),
        ],
        out_specs=pl.BlockSpec((BM, D), lambda i: (i, 0)),
        out_shape=jax.ShapeDtypeStruct((N_NODES, D), jnp.float32),
    )(x, agg, agg, W, b.reshape(1, D))
    return out


# speedup vs baseline: 1.1791x; 1.1791x over previous
import functools

import jax
import jax.numpy as jnp
from jax import lax
from jax.experimental import pallas as pl
from jax.experimental.pallas import tpu as pltpu
from jax.experimental.pallas import tpu_sc as plsc

N_NODES = 10000
N_EDGES = 320000
D = 128

NC = 2
NS = 16
NW = NC * NS

CHUNK = 128
NROWS = N_EDGES // CHUNK
CHUNKS_T = NROWS // NW
XBASE = NW * CHUNKS_T
NEXTRA = NROWS - XBASE
IBLK = 16
BLOCKS = [IBLK] * (CHUNKS_T // IBLK) + (
    [CHUNKS_T % IBLK] if CHUNKS_T % IBLK else [])
N_PAD = 10112
ZROWS = N_PAD // NS


def _sc_aggregate(edges3, x):
    mesh = plsc.VectorSubcoreMesh(core_axis_name="c", subcore_axis_name="s")

    @functools.partial(
        pl.kernel,
        out_type=jax.ShapeDtypeStruct((NC * N_PAD, D), jnp.float32),
        mesh=mesh,
        compiler_params=pltpu.CompilerParams(use_tc_tiling_on_sc=False),
        scratch_types=[
            pltpu.VMEM((2, IBLK, CHUNK), jnp.int32),
            pltpu.VMEM((2, IBLK, CHUNK), jnp.int32),
            pltpu.VMEM((CHUNK, D), jnp.float32),
            pltpu.VMEM((CHUNK, D), jnp.float32),
            pltpu.VMEM_SHARED((N_PAD, D), jnp.float32),
            pltpu.SemaphoreType.DMA,
            pltpu.SemaphoreType.DMA,
            pltpu.SemaphoreType.DMA,
        ],
    )
    def sc_kernel(e_hbm, x_hbm, out_hbm,
                  src_v, dst_v, bufa, bufb, agg, sema, semb, semi):
        cid = lax.axis_index("c")
        sid = lax.axis_index("s")
        tid = cid * NS + sid
        src_hbm = e_hbm.at[0]
        dst_hbm = e_hbm.at[1]

        @pl.loop(0, CHUNK)
        def _(i):
            for k in range(D // 16):
                bufa[i, pl.ds(k * 16, 16)] = jnp.zeros((16,), jnp.float32)

        zbase = sid * ZROWS
        nfull = ZROWS // CHUNK
        for z in range(nfull):
            pltpu.sync_copy(bufa, agg.at[pl.ds(zbase + z * CHUNK, CHUNK)])
        rem = ZROWS - nfull * CHUNK
        if rem:
            pltpu.sync_copy(bufa.at[pl.ds(0, rem)],
                            agg.at[pl.ds(zbase + nfull * CHUNK, rem)])
        plsc.subcore_barrier()

        base = tid * CHUNKS_T
        pltpu.sync_copy(src_hbm.at[pl.ds(base, IBLK)], src_v.at[0])
        pltpu.sync_copy(dst_hbm.at[pl.ds(base, IBLK)], dst_v.at[0])

        for blk, blen in enumerate(BLOCKS):
            cur = blk % 2
            nxt = 1 - cur
            if blk + 1 < len(BLOCKS):
                hs = pltpu.async_copy(
                    src_hbm.at[pl.ds(base + (blk + 1) * IBLK, IBLK)],
                    src_v.at[nxt], semi)
                hd = pltpu.async_copy(
                    dst_hbm.at[pl.ds(base + (blk + 1) * IBLK, IBLK)],
                    dst_v.at[nxt], semi)
            sv = src_v.at[cur]
            dv = dst_v.at[cur]
            pltpu.async_copy(x_hbm.at[sv.at[0]], bufa, sema)

            @pl.loop(0, blen // 2)
            def _(g):
                j0 = g * 2
                j1 = j0 + 1
                pltpu.async_copy(x_hbm.at[sv.at[j1]], bufb, semb)
                pltpu.make_async_copy(x_hbm.at[sv.at[j0]], bufa, sema).wait()
                pltpu.sync_copy(bufa, agg.at[dv.at[j0]], add=True)

                @pl.when(j1 + 1 < blen)
                def _():
                    pltpu.async_copy(x_hbm.at[sv.at[j1 + 1]], bufa, sema)

                pltpu.make_async_copy(x_hbm.at[sv.at[j1]], bufb, semb).wait()
                pltpu.sync_copy(bufb, agg.at[dv.at[j1]], add=True)

            if blk + 1 < len(BLOCKS):
                hs.wait()
                hd.wait()

        @pl.when(tid < NEXTRA)
        def _():
            pltpu.sync_copy(src_hbm.at[pl.ds(XBASE, NEXTRA)],
                            src_v.at[0, pl.ds(0, NEXTRA)])
            pltpu.sync_copy(dst_hbm.at[pl.ds(XBASE, NEXTRA)],
                            dst_v.at[0, pl.ds(0, NEXTRA)])
            pltpu.sync_copy(x_hbm.at[src_v.at[0, tid]], bufa)
            pltpu.sync_copy(bufa, agg.at[dst_v.at[0, tid]], add=True)

        plsc.subcore_barrier()

        obase = sid * ZROWS
        pltpu.sync_copy(agg.at[pl.ds(obase, ZROWS)],
                        out_hbm.at[pl.ds(cid * N_PAD + obase, ZROWS)])

    return sc_kernel(edges3, x)


def _tc_body(x_ref, a0_ref, a1_ref, w_ref, b_ref, o_ref):
    h = x_ref[...] + a0_ref[0] + a1_ref[0]
    o_ref[...] = lax.dot_general(
        h, w_ref[...],
        dimension_numbers=(((1,), (1,)), ((), ())),
        preferred_element_type=jnp.float32,
    ) + b_ref[...]


def kernel(x, edge_index, W, b):
    edges3 = edge_index.reshape(2, NROWS, CHUNK)

    agg = _sc_aggregate(edges3, x).reshape(NC, N_PAD, D)

    BM = 1000
    nb = N_NODES // BM
    out = pl.pallas_call(
        _tc_body,
        grid=(nb,),
        in_specs=[
            pl.BlockSpec((BM, D), lambda i: (i, 0)),
            pl.BlockSpec((1, BM, D), lambda i: (0, i, 0)),
            pl.BlockSpec((1, BM, D), lambda i: (1, i, 0)),
            pl.BlockSpec((D, D), lambda i: (0, 0)),
            pl.BlockSpec((1, D), lambda i: (0, 0)),
        ],
        out_specs=pl.BlockSpec((BM, D), lambda i: (i, 0)),
        out_shape=jax.ShapeDtypeStruct((N_NODES, D), jnp.float32),
    )(x, agg, agg, W, b.reshape(1, D))
    return out
